# baseline (speedup 1.0000x reference)
import jax
import jax.numpy as jnp
from jax.experimental import pallas as pl
from jax.experimental.pallas import tpu as pltpu

_T, _E, _D_MODEL, _D_FF = 128, 16, 768, 1536
_EB = 2
_NSTEPS = _E // _EB


def _moe_kernel(ids_ref, x_ref, w1_ref, w3_ref, w2_ref, out_ref):
    s = pl.program_id(0)

    acc = jnp.zeros((_T, _D_MODEL), jnp.float32)
    for k in range(_EB):
        e = s * _EB + k
        mask = ids_ref[...] == e
        xm = jnp.where(mask, x_ref[...], 0.0)

        g = jax.lax.dot_general(xm, w1_ref[k], (((1,), (1,)), ((), ())),
                                preferred_element_type=jnp.float32)
        u = jax.lax.dot_general(xm, w3_ref[k], (((1,), (1,)), ((), ())),
                                preferred_element_type=jnp.float32)
        h = (g * jax.nn.sigmoid(g)) * u
        acc += jax.lax.dot_general(h, w2_ref[k], (((1,), (1,)), ((), ())),
                                   preferred_element_type=jnp.float32)

    @pl.when(s == 0)
    def _init():
        out_ref[...] = jnp.zeros_like(out_ref)

    out_ref[...] += acc


def kernel(x, token_expert_ids, w1, w3, w2):
    ids = token_expert_ids.astype(jnp.int32).reshape(_T, 1)
    return pl.pallas_call(
        _moe_kernel,
        grid=(_NSTEPS,),
        in_specs=[
            pl.BlockSpec((_T, 1), lambda s: (0, 0)),
            pl.BlockSpec((_T, _D_MODEL), lambda s: (0, 0)),
            pl.BlockSpec((_EB, _D_FF, _D_MODEL), lambda s: (s, 0, 0)),
            pl.BlockSpec((_EB, _D_FF, _D_MODEL), lambda s: (s, 0, 0)),
            pl.BlockSpec((_EB, _D_MODEL, _D_FF), lambda s: (s, 0, 0)),
        ],
        out_specs=pl.BlockSpec((_T, _D_MODEL), lambda s: (0, 0)),
        out_shape=jax.ShapeDtypeStruct((_T, _D_MODEL), jnp.float32),
        compiler_params=pltpu.CompilerParams(
            dimension_semantics=("arbitrary",),
            vmem_limit_bytes=100 * 1024 * 1024,
        ),
    )(ids, x, w1, w3, w2)
